# Initial kernel scaffold; baseline (speedup 1.0000x reference)
import jax
import jax.numpy as jnp
from jax.experimental import pallas as pl


def kernel(x, grid, edge_features, proj_w1, proj_b1, proj_w2, proj_b2, blk_w, blk_b, dec_w1, dec_b1, dec_w2, dec_b2, edge_index):
    raise NotImplementedError("write your pallas kernel here")


# speedup vs baseline: 30.7345x; 30.7345x over previous
import functools

import jax
import jax.numpy as jnp
from jax import lax
from jax.experimental import pallas as pl
from jax.experimental.pallas import tpu as pltpu
from jax.experimental.pallas import tpu_sc as plsc

N = 100000
E = 3200000
LATENT = 16

NC = 2
NS = 16
NW = NC * NS

CW = 125
ROWS = E // CW
RW = ROWS // NW
IB = 16
NBLK = RW // IB
NPRE = 4
NP = 100096
NT = NP // NS
ZR = 368


def _gelu(t):
    return 0.5 * t * (1.0 + lax.erf(t * (2.0 ** -0.5)))


def _sc_body(h_ref, src_ref, dst_ref, out_ref,
             aggr, sbuf0, dbuf0, sbuf1, dbuf1, rb0, rb1, rb2, rb3, zbuf,
             sem_s0, sem_d0, sem_s1, sem_d1, g0, g1, g2, g3):
    c = lax.axis_index("c")
    s = lax.axis_index("s")
    wid = c * NS + s
    row0 = wid * RW
    rbuf = [rb0, rb1, rb2, rb3]
    gsem = [g0, g1, g2, g3]

    def fire_idx(r, sb, db, ss, sd):
        pltpu.async_copy(src_ref.at[pl.ds(r, IB)], sb, ss)
        pltpu.async_copy(dst_ref.at[pl.ds(r, IB)], db, sd)

    def wait_idx(sb, db, ss, sd):
        pltpu.make_async_copy(src_ref.at[pl.ds(0, IB)], sb, ss).wait()
        pltpu.make_async_copy(dst_ref.at[pl.ds(0, IB)], db, sd).wait()

    fire_idx(row0, sbuf0, dbuf0, sem_s0, sem_d0)

    @pl.loop(0, ZR)
    def _(i):
        zbuf[i] = jnp.zeros((LATENT,), jnp.float32)

    for k in range(NT // ZR):
        pltpu.sync_copy(zbuf, aggr.at[pl.ds(s * NT + k * ZR, ZR)])
    plsc.subcore_barrier()

    def process(sb, db):
        descs = []
        for j in range(NPRE):
            descs.append(pltpu.async_copy(h_ref.at[sb.at[j]], rbuf[j], gsem[j]))
        for j in range(IB):
            q = j % NPRE
            descs[q].wait()
            pltpu.sync_copy(rbuf[q], aggr.at[db.at[j]], add=True)
            nj = j + NPRE
            if nj < IB:
                descs[q] = pltpu.async_copy(h_ref.at[sb.at[nj]], rbuf[q], gsem[q])

    @pl.loop(0, NBLK, step=2)
    def _(b0):
        fire_idx(row0 + (b0 + 1) * IB, sbuf1, dbuf1, sem_s1, sem_d1)
        wait_idx(sbuf0, dbuf0, sem_s0, sem_d0)
        process(sbuf0, dbuf0)

        @pl.when(b0 + 2 < NBLK)
        def _():
            fire_idx(row0 + (b0 + 2) * IB, sbuf0, dbuf0, sem_s0, sem_d0)

        wait_idx(sbuf1, dbuf1, sem_s1, sem_d1)
        process(sbuf1, dbuf1)

    plsc.subcore_barrier()
    pltpu.sync_copy(aggr.at[pl.ds(s * NT, NT)], out_ref.at[c].at[pl.ds(s * NT, NT)])


def _sc_aggregate(h, src2d, dst2d):
    mesh = plsc.VectorSubcoreMesh(core_axis_name="c", subcore_axis_name="s",
                                  num_cores=NC, num_subcores=NS)
    f = pl.kernel(
        _sc_body,
        out_type=jax.ShapeDtypeStruct((NC, NP, LATENT), jnp.float32),
        mesh=mesh,
        compiler_params=pltpu.CompilerParams(use_tc_tiling_on_sc=False),
        scratch_types=[
            pltpu.VMEM_SHARED((NP, LATENT), jnp.float32),
            pltpu.VMEM((IB, CW), jnp.int32),
            pltpu.VMEM((IB, CW), jnp.int32),
            pltpu.VMEM((IB, CW), jnp.int32),
            pltpu.VMEM((IB, CW), jnp.int32),
            pltpu.VMEM((CW, LATENT), jnp.float32),
            pltpu.VMEM((CW, LATENT), jnp.float32),
            pltpu.VMEM((CW, LATENT), jnp.float32),
            pltpu.VMEM((CW, LATENT), jnp.float32),
            pltpu.VMEM((ZR, LATENT), jnp.float32),
            pltpu.SemaphoreType.DMA,
            pltpu.SemaphoreType.DMA,
            pltpu.SemaphoreType.DMA,
            pltpu.SemaphoreType.DMA,
            pltpu.SemaphoreType.DMA,
            pltpu.SemaphoreType.DMA,
            pltpu.SemaphoreType.DMA,
            pltpu.SemaphoreType.DMA,
        ],
    )
    return f(h, src2d, dst2d)


_RB = 4000


def _proj_body(x_ref, g_ref, w1x_ref, w1g_ref, b1_ref, w2_ref, b2_ref, o_ref):
    pre = (jnp.dot(x_ref[...], w1x_ref[...], preferred_element_type=jnp.float32)
           + jnp.dot(g_ref[...], w1g_ref[...], preferred_element_type=jnp.float32)
           + b1_ref[...])
    o_ref[...] = (jnp.dot(_gelu(pre), w2_ref[...],
                          preferred_element_type=jnp.float32) + b2_ref[...])


def _project(x, grid, w1, b1, w2, b2):
    w1x, w1g = w1[:x.shape[1]], w1[x.shape[1]:]
    nb = N // _RB
    return pl.pallas_call(
        _proj_body,
        grid=(nb,),
        in_specs=[
            pl.BlockSpec((_RB, x.shape[1]), lambda i: (i, 0)),
            pl.BlockSpec((_RB, 2), lambda i: (i, 0)),
            pl.BlockSpec(w1x.shape, lambda i: (0, 0)),
            pl.BlockSpec(w1g.shape, lambda i: (0, 0)),
            pl.BlockSpec((1, LATENT), lambda i: (0, 0)),
            pl.BlockSpec((LATENT, LATENT), lambda i: (0, 0)),
            pl.BlockSpec((1, LATENT), lambda i: (0, 0)),
        ],
        out_specs=pl.BlockSpec((_RB, LATENT), lambda i: (i, 0)),
        out_shape=jax.ShapeDtypeStruct((N, LATENT), jnp.float32),
    )(x, grid, w1x, w1g, b1.reshape(1, -1), w2, b2.reshape(1, -1))


def _update_body(h_ref, p_ref, bw_ref, bb_ref, d1_ref, db1_ref, d2t_ref,
                 db2_ref, o_ref):
    t = _gelu(jnp.dot(h_ref[...], bw_ref[...], preferred_element_type=jnp.float32)
              + bb_ref[...] + p_ref[0] + p_ref[1])
    m = _gelu(jnp.dot(t, d1_ref[...], preferred_element_type=jnp.float32)
              + db1_ref[...])
    o_ref[...] = (jnp.sum(m * d2t_ref[...], axis=1, keepdims=True)
                  + db2_ref[...])


def _update_decode(h, part, blk_w, blk_b, dec_w1, dec_b1, dec_w2, dec_b2):
    nb = N // _RB
    return pl.pallas_call(
        _update_body,
        grid=(nb,),
        in_specs=[
            pl.BlockSpec((_RB, LATENT), lambda i: (i, 0)),
            pl.BlockSpec((NC, _RB, LATENT), lambda i: (0, i, 0)),
            pl.BlockSpec((LATENT, LATENT), lambda i: (0, 0)),
            pl.BlockSpec((1, LATENT), lambda i: (0, 0)),
            pl.BlockSpec((LATENT, LATENT), lambda i: (0, 0)),
            pl.BlockSpec((1, LATENT), lambda i: (0, 0)),
            pl.BlockSpec((1, LATENT), lambda i: (0, 0)),
            pl.BlockSpec((1, 1), lambda i: (0, 0)),
        ],
        out_specs=pl.BlockSpec((_RB, 1), lambda i: (i, 0)),
        out_shape=jax.ShapeDtypeStruct((N, 1), jnp.float32),
    )(h, part, blk_w, blk_b.reshape(1, -1), dec_w1, dec_b1.reshape(1, -1),
      dec_w2.reshape(1, -1), dec_b2.reshape(1, 1))


def kernel(x, grid, edge_features, proj_w1, proj_b1, proj_w2, proj_b2,
           blk_w, blk_b, dec_w1, dec_b1, dec_w2, dec_b2, edge_index):
    del edge_features
    src2d = edge_index[0].reshape(ROWS, CW)
    dst2d = edge_index[1].reshape(ROWS, CW)
    h = _project(x, grid, proj_w1, proj_b1, proj_w2, proj_b2)
    part = _sc_aggregate(h, src2d, dst2d)
    return _update_decode(h, part, blk_w, blk_b, dec_w1, dec_b1,
                          dec_w2, dec_b2)


# speedup vs baseline: 64.9851x; 2.1144x over previous
import jax
import jax.numpy as jnp
from jax import lax
from jax.experimental import pallas as pl
from jax.experimental.pallas import tpu as pltpu
from jax.experimental.pallas import tpu_sc as plsc

N = 100000
E = 3200000
LATENT = 16

NC = 2
NS = 16
NW = NC * NS

CW = 128
CHUNKS = E // CW
CPW = CHUNKS // NW
IB = 16
NFULL = 48
NB = 6
GLA = 3
NP = 100096
NT = NP // NS
ZR = 184


def _gelu(t):
    return 0.5 * t * (1.0 + lax.erf(t * (2.0 ** -0.5)))


def _sc_body(h_ref, e_ref, out_ref,
             aggr, ib0, ib1, rb0, rb1, rb2, rb3, rb4, rb5, zbuf,
             isem0, isem1, g0, g1, g2, g3, g4, g5, s0, s1, s2, s3, s4, s5):
    c = lax.axis_index("c")
    s = lax.axis_index("s")
    wid = c * NS + s
    extra = (wid < 8).astype(jnp.int32)
    base = wid * CPW + jnp.minimum(wid, 8)
    count = CPW + extra
    rb = [rb0, rb1, rb2, rb3, rb4, rb5]
    gsem = [g0, g1, g2, g3, g4, g5]
    ssem = [s0, s1, s2, s3, s4, s5]

    def fire_idx(chunk0, ib, sem):
        pltpu.async_copy(e_ref.at[pl.ds(chunk0, IB)], ib, sem)

    def wait_idx(ib, sem):
        pltpu.make_async_copy(e_ref.at[pl.ds(0, IB)], ib, sem).wait()

    fire_idx(base, ib0, isem0)

    @pl.loop(0, ZR)
    def _(i):
        zbuf[i] = jnp.zeros((LATENT,), jnp.float32)

    for k in range(NT // ZR):
        pltpu.sync_copy(zbuf, aggr.at[pl.ds(s * NT + k * ZR, ZR)])
    plsc.subcore_barrier()

    def process16(ib):
        dg = [None] * NB
        ds = [None] * NB
        for t in range(IB + GLA):
            jg = t
            js = t - GLA
            if jg < IB:
                q = jg % NB
                if jg >= NB:
                    ds[q].wait()
                dg[q] = pltpu.async_copy(h_ref.at[ib.at[jg, 0]], rb[q], gsem[q])
            if 0 <= js < IB:
                q = js % NB
                dg[q].wait()
                ds[q] = pltpu.async_copy(rb[q], aggr.at[ib.at[js, 1]],
                                         ssem[q], add=True)
        for js in range(IB - NB, IB):
            ds[js % NB].wait()

    @pl.loop(0, NFULL, step=2)
    def _(b0):
        fire_idx(base + (b0 + 1) * IB, ib1, isem1)
        wait_idx(ib0, isem0)
        process16(ib0)

        @pl.when(b0 + 2 < NFULL)
        def _():
            fire_idx(base + (b0 + 2) * IB, ib0, isem0)

        wait_idx(ib1, isem1)
        process16(ib1)

    rem = count - NFULL * IB
    fire_idx(base + count - IB, ib0, isem0)
    wait_idx(ib0, isem0)
    for j in range(IB):
        @pl.when(j >= IB - rem)
        def _():
            pltpu.async_copy(h_ref.at[ib0.at[j, 0]], rb0, g0).wait()
            pltpu.sync_copy(rb0, aggr.at[ib0.at[j, 1]], add=True)

    plsc.subcore_barrier()
    pltpu.sync_copy(aggr.at[pl.ds(s * NT, NT)],
                    out_ref.at[c].at[pl.ds(s * NT, NT)])


def _sc_aggregate(h, e3):
    mesh = plsc.VectorSubcoreMesh(core_axis_name="c", subcore_axis_name="s",
                                  num_cores=NC, num_subcores=NS)
    f = pl.kernel(
        _sc_body,
        out_type=jax.ShapeDtypeStruct((NC, NP, LATENT), jnp.float32),
        mesh=mesh,
        compiler_params=pltpu.CompilerParams(use_tc_tiling_on_sc=False),
        scratch_types=(
            [pltpu.VMEM_SHARED((NP, LATENT), jnp.float32)]
            + [pltpu.VMEM((IB, 2, CW), jnp.int32)] * 2
            + [pltpu.VMEM((CW, LATENT), jnp.float32)] * NB
            + [pltpu.VMEM((ZR, LATENT), jnp.float32)]
            + [pltpu.SemaphoreType.DMA] * (2 + 2 * NB)
        ),
    )
    return f(h, e3)


_RB = 512


def _proj_body(xg_ref, w1_ref, b1_ref, w2_ref, b2_ref, o_ref):
    pre = (jnp.dot(xg_ref[...], w1_ref[...], preferred_element_type=jnp.float32)
           + b1_ref[...])
    o_ref[...] = (jnp.dot(_gelu(pre), w2_ref[...],
                          preferred_element_type=jnp.float32) + b2_ref[...])


def _project_packed(xgp, w1p, b1p, w2p, b2p):
    nb = pl.cdiv(N // 8, _RB)
    return pl.pallas_call(
        _proj_body,
        grid=(nb,),
        in_specs=[
            pl.BlockSpec((_RB, 96), lambda i: (i, 0)),
            pl.BlockSpec((96, 128), lambda i: (0, 0)),
            pl.BlockSpec((1, 128), lambda i: (0, 0)),
            pl.BlockSpec((128, 128), lambda i: (0, 0)),
            pl.BlockSpec((1, 128), lambda i: (0, 0)),
        ],
        out_specs=pl.BlockSpec((_RB, 128), lambda i: (i, 0)),
        out_shape=jax.ShapeDtypeStruct((N // 8, 128), jnp.float32),
    )(xgp, w1p, b1p, w2p, b2p)


def _update_body(h_ref, p_ref, bw_ref, bb_ref, d1_ref, db1_ref, w2t_ref,
                 sel_ref, db2_ref, o_ref):
    t = _gelu(jnp.dot(h_ref[...], bw_ref[...], preferred_element_type=jnp.float32)
              + bb_ref[...] + p_ref[0] + p_ref[1])
    m = _gelu(jnp.dot(t, d1_ref[...], preferred_element_type=jnp.float32)
              + db1_ref[...])
    o_ref[...] = (jnp.dot(m * w2t_ref[...], sel_ref[...],
                          preferred_element_type=jnp.float32) + db2_ref[...])


def _update_packed(hp, pp, bwp, bbp, d1p, db1p, w2t, sel, db2):
    nb = pl.cdiv(N // 8, _RB)
    return pl.pallas_call(
        _update_body,
        grid=(nb,),
        in_specs=[
            pl.BlockSpec((_RB, 128), lambda i: (i, 0)),
            pl.BlockSpec((NC, _RB, 128), lambda i: (0, i, 0)),
            pl.BlockSpec((128, 128), lambda i: (0, 0)),
            pl.BlockSpec((1, 128), lambda i: (0, 0)),
            pl.BlockSpec((128, 128), lambda i: (0, 0)),
            pl.BlockSpec((1, 128), lambda i: (0, 0)),
            pl.BlockSpec((1, 128), lambda i: (0, 0)),
            pl.BlockSpec((128, 8), lambda i: (0, 0)),
            pl.BlockSpec((1, 8), lambda i: (0, 0)),
        ],
        out_specs=pl.BlockSpec((_RB, 8), lambda i: (i, 0)),
        out_shape=jax.ShapeDtypeStruct((N // 8, 8), jnp.float32),
    )(hp, pp, bwp, bbp, d1p, db1p, w2t, sel, db2)


def kernel(x, grid, edge_features, proj_w1, proj_b1, proj_w2, proj_b2,
           blk_w, blk_b, dec_w1, dec_b1, dec_w2, dec_b2, edge_index):
    del edge_features
    f32 = jnp.float32
    eye8 = jnp.eye(8, dtype=f32)

    xg = jnp.concatenate([x, grid], axis=1)
    xgp = xg.reshape(N // 8, 96)
    w1p = jnp.kron(eye8, proj_w1)
    w2p = jnp.kron(eye8, proj_w2)
    hp = _project_packed(xgp, w1p, jnp.tile(proj_b1, 8).reshape(1, 128),
                         w2p, jnp.tile(proj_b2, 8).reshape(1, 128))

    h_sc = hp.reshape(N, LATENT)
    e3 = edge_index.reshape(2, CHUNKS, CW).transpose(1, 0, 2)
    part = _sc_aggregate(h_sc, e3)
    pp = part.reshape(NC, NP // 8, 128)

    bwp = jnp.kron(eye8, blk_w)
    d1p = jnp.kron(eye8, dec_w1)
    w2t = jnp.tile(dec_w2[:, 0], 8).reshape(1, 128)
    sel = (jnp.arange(128)[:, None] // 16 ==
           jnp.arange(8)[None, :]).astype(f32)
    op = _update_packed(hp, pp, bwp, jnp.tile(blk_b, 8).reshape(1, 128),
                        d1p, jnp.tile(dec_b1, 8).reshape(1, 128),
                        w2t, sel, jnp.broadcast_to(dec_b2, (8,)).reshape(1, 8))
    return op.reshape(N, 1)


# speedup vs baseline: 66.9522x; 1.0303x over previous
import jax
import jax.numpy as jnp
from jax import lax
from jax.experimental import pallas as pl
from jax.experimental.pallas import tpu as pltpu
from jax.experimental.pallas import tpu_sc as plsc

N = 100000
E = 3200000
LATENT = 16

NC = 2
NS = 16
NW = NC * NS

CW = 128
CHUNKS = E // CW
CPW = CHUNKS // NW
IB = 16
NFULL = 48
NB = 8
GLA = 4
NP = 100096
NT = NP // NS
ZR = 184


def _gelu(t):
    return 0.5 * t * (1.0 + lax.erf(t * (2.0 ** -0.5)))


def _sc_body(h_ref, e_ref, out_ref, aggr, ib0, ib1, *rest):
    rb = list(rest[:NB])
    zbuf = rest[NB]
    isem0, isem1 = rest[NB + 1], rest[NB + 2]
    gsem = list(rest[NB + 3:NB + 3 + NB])
    ssem = list(rest[NB + 3 + NB:NB + 3 + 2 * NB])
    c = lax.axis_index("c")
    s = lax.axis_index("s")
    wid = c * NS + s
    extra = (wid < 8).astype(jnp.int32)
    base = wid * CPW + jnp.minimum(wid, 8)
    count = CPW + extra

    def fire_idx(chunk0, ib, sem):
        pltpu.async_copy(e_ref.at[pl.ds(chunk0, IB)], ib, sem)

    def wait_idx(ib, sem):
        pltpu.make_async_copy(e_ref.at[pl.ds(0, IB)], ib, sem).wait()

    fire_idx(base, ib0, isem0)

    @pl.loop(0, ZR)
    def _(i):
        zbuf[i] = jnp.zeros((LATENT,), jnp.float32)

    for k in range(NT // ZR):
        pltpu.sync_copy(zbuf, aggr.at[pl.ds(s * NT + k * ZR, ZR)])
    plsc.subcore_barrier()

    def process16(ib):
        dg = [None] * NB
        ds = [None] * NB
        for t in range(IB + GLA):
            jg = t
            js = t - GLA
            if jg < IB:
                q = jg % NB
                if jg >= NB:
                    ds[q].wait()
                dg[q] = pltpu.async_copy(h_ref.at[ib.at[jg, 0]], rb[q], gsem[q])
            if 0 <= js < IB:
                q = js % NB
                dg[q].wait()
                ds[q] = pltpu.async_copy(rb[q], aggr.at[ib.at[js, 1]],
                                         ssem[q], add=True)
        for js in range(IB - NB, IB):
            ds[js % NB].wait()

    @pl.loop(0, NFULL, step=2)
    def _(b0):
        fire_idx(base + (b0 + 1) * IB, ib1, isem1)
        wait_idx(ib0, isem0)
        process16(ib0)

        @pl.when(b0 + 2 < NFULL)
        def _():
            fire_idx(base + (b0 + 2) * IB, ib0, isem0)

        wait_idx(ib1, isem1)
        process16(ib1)

    rem = count - NFULL * IB
    fire_idx(base + count - IB, ib0, isem0)
    wait_idx(ib0, isem0)
    for j in range(IB):
        @pl.when(j >= IB - rem)
        def _():
            pltpu.async_copy(h_ref.at[ib0.at[j, 0]], rb[0], gsem[0]).wait()
            pltpu.sync_copy(rb[0], aggr.at[ib0.at[j, 1]], add=True)

    plsc.subcore_barrier()
    pltpu.sync_copy(aggr.at[pl.ds(s * NT, NT)],
                    out_ref.at[c].at[pl.ds(s * NT, NT)])


def _sc_aggregate(h, e3):
    mesh = plsc.VectorSubcoreMesh(core_axis_name="c", subcore_axis_name="s",
                                  num_cores=NC, num_subcores=NS)
    f = pl.kernel(
        _sc_body,
        out_type=jax.ShapeDtypeStruct((NC, NP, LATENT), jnp.float32),
        mesh=mesh,
        compiler_params=pltpu.CompilerParams(use_tc_tiling_on_sc=False),
        scratch_types=(
            [pltpu.VMEM_SHARED((NP, LATENT), jnp.float32)]
            + [pltpu.VMEM((IB, 2, CW), jnp.int32)] * 2
            + [pltpu.VMEM((CW, LATENT), jnp.float32)] * NB
            + [pltpu.VMEM((ZR, LATENT), jnp.float32)]
            + [pltpu.SemaphoreType.DMA] * (2 + 2 * NB)
        ),
    )
    return f(h, e3)


_RB = 512


def _proj_body(xg_ref, w1_ref, b1_ref, w2_ref, b2_ref, o_ref):
    pre = (jnp.dot(xg_ref[...], w1_ref[...], preferred_element_type=jnp.float32)
           + b1_ref[...])
    o_ref[...] = (jnp.dot(_gelu(pre), w2_ref[...],
                          preferred_element_type=jnp.float32) + b2_ref[...])


def _project_packed(xgp, w1p, b1p, w2p, b2p):
    nb = pl.cdiv(N // 8, _RB)
    return pl.pallas_call(
        _proj_body,
        grid=(nb,),
        in_specs=[
            pl.BlockSpec((_RB, 96), lambda i: (i, 0)),
            pl.BlockSpec((96, 128), lambda i: (0, 0)),
            pl.BlockSpec((1, 128), lambda i: (0, 0)),
            pl.BlockSpec((128, 128), lambda i: (0, 0)),
            pl.BlockSpec((1, 128), lambda i: (0, 0)),
        ],
        out_specs=pl.BlockSpec((_RB, 128), lambda i: (i, 0)),
        out_shape=jax.ShapeDtypeStruct((N // 8, 128), jnp.float32),
    )(xgp, w1p, b1p, w2p, b2p)


def _update_body(h_ref, p_ref, bw_ref, bb_ref, d1_ref, db1_ref, w2t_ref,
                 sel_ref, db2_ref, o_ref):
    t = _gelu(jnp.dot(h_ref[...], bw_ref[...], preferred_element_type=jnp.float32)
              + bb_ref[...] + p_ref[0] + p_ref[1])
    m = _gelu(jnp.dot(t, d1_ref[...], preferred_element_type=jnp.float32)
              + db1_ref[...])
    o_ref[...] = (jnp.dot(m * w2t_ref[...], sel_ref[...],
                          preferred_element_type=jnp.float32) + db2_ref[...])


def _update_packed(hp, pp, bwp, bbp, d1p, db1p, w2t, sel, db2):
    nb = pl.cdiv(N // 8, _RB)
    return pl.pallas_call(
        _update_body,
        grid=(nb,),
        in_specs=[
            pl.BlockSpec((_RB, 128), lambda i: (i, 0)),
            pl.BlockSpec((NC, _RB, 128), lambda i: (0, i, 0)),
            pl.BlockSpec((128, 128), lambda i: (0, 0)),
            pl.BlockSpec((1, 128), lambda i: (0, 0)),
            pl.BlockSpec((128, 128), lambda i: (0, 0)),
            pl.BlockSpec((1, 128), lambda i: (0, 0)),
            pl.BlockSpec((1, 128), lambda i: (0, 0)),
            pl.BlockSpec((128, 8), lambda i: (0, 0)),
            pl.BlockSpec((1, 8), lambda i: (0, 0)),
        ],
        out_specs=pl.BlockSpec((_RB, 8), lambda i: (i, 0)),
        out_shape=jax.ShapeDtypeStruct((N // 8, 8), jnp.float32),
    )(hp, pp, bwp, bbp, d1p, db1p, w2t, sel, db2)


def kernel(x, grid, edge_features, proj_w1, proj_b1, proj_w2, proj_b2,
           blk_w, blk_b, dec_w1, dec_b1, dec_w2, dec_b2, edge_index):
    del edge_features
    f32 = jnp.float32
    eye8 = jnp.eye(8, dtype=f32)

    xgt = jnp.concatenate([x, grid], axis=1).T
    xgp = lax.reshape(xgt, (N // 8, 96), dimensions=(1, 0))
    w1p = jnp.kron(eye8, proj_w1)
    w2p = jnp.kron(eye8, proj_w2)
    hp = _project_packed(xgp, w1p, jnp.tile(proj_b1, 8).reshape(1, 128),
                         w2p, jnp.tile(proj_b2, 8).reshape(1, 128))

    h_sc = hp.reshape(N, LATENT)
    e3 = edge_index.reshape(2, CHUNKS, CW).transpose(1, 0, 2)
    part = _sc_aggregate(h_sc, e3)
    pp = part.reshape(NC, NP // 8, 128)

    bwp = jnp.kron(eye8, blk_w)
    d1p = jnp.kron(eye8, dec_w1)
    w2t = jnp.tile(dec_w2[:, 0], 8).reshape(1, 128)
    sel = (jnp.arange(128)[:, None] // 16 ==
           jnp.arange(8)[None, :]).astype(f32)
    op = _update_packed(hp, pp, bwp, jnp.tile(blk_b, 8).reshape(1, 128),
                        d1p, jnp.tile(dec_b1, 8).reshape(1, 128),
                        w2t, sel, jnp.broadcast_to(dec_b2, (8,)).reshape(1, 8))
    return op.reshape(N, 1)


# speedup vs baseline: 68.9713x; 1.0302x over previous
import jax
import jax.numpy as jnp
from jax import lax
from jax.experimental import pallas as pl
from jax.experimental.pallas import tpu as pltpu
from jax.experimental.pallas import tpu_sc as plsc

N = 100000
E = 3200000
LATENT = 16

NC = 2
NS = 16
NW = NC * NS

SEG = 12544
NP = 8 * SEG

CW = 128
CHUNKS = E // CW
CPW = CHUNKS // NW
IB = 16
NFULL = 48
NB = 8
GLA = 4
NT = NP // NS
ZR = 224


def _gelu(t):
    return 0.5 * t * (1.0 + lax.erf(t * (2.0 ** -0.5)))


def _remap(n):
    a = ((n >> 7) * 669) >> 16
    return ((n - a * SEG) << 3) + a


def _remap_body(e_ref, o_ref):
    o_ref[...] = _remap(e_ref[...])


def _edge_remap(ei_lin):
    nb = 25
    rows = 2 * CHUNKS
    return pl.pallas_call(
        _remap_body,
        grid=(nb,),
        in_specs=[pl.BlockSpec((rows // nb, CW), lambda i: (i, 0))],
        out_specs=pl.BlockSpec((rows // nb, CW), lambda i: (i, 0)),
        out_shape=jax.ShapeDtypeStruct((rows, CW), jnp.int32),
    )(ei_lin)


_PB = 256


def _proj_body(*refs):
    xrefs = refs[:8]
    w1_ref, b1_ref, w2_ref, b2_ref, o_ref = refs[8:]
    dn = (((0,), (0,)), ((), ()))
    parts = []
    for a in range(8):
        pre = (lax.dot_general(xrefs[a][...], w1_ref[...], dn,
                               preferred_element_type=jnp.float32)
               + b1_ref[...])
        h2 = (jnp.dot(_gelu(pre), w2_ref[...],
                      preferred_element_type=jnp.float32) + b2_ref[...])
        if a == 7:
            gr = (_PB * pl.program_id(0)
                  + lax.broadcasted_iota(jnp.int32, (_PB, 1), 0))
            h2 = jnp.where(gr < N - 7 * SEG, h2, 0.0)
        parts.append(h2)
    o_ref[...] = jnp.concatenate(parts, axis=1)


def _project_packed(xgt, w1, b1, w2, b2):
    specs = [
        pl.BlockSpec((12, _PB), (lambda i, a=a: (0, (SEG // _PB) * a + i)))
        for a in range(8)
    ]
    return pl.pallas_call(
        _proj_body,
        grid=(SEG // _PB,),
        in_specs=specs + [
            pl.BlockSpec((12, LATENT), lambda i: (0, 0)),
            pl.BlockSpec((1, LATENT), lambda i: (0, 0)),
            pl.BlockSpec((LATENT, LATENT), lambda i: (0, 0)),
            pl.BlockSpec((1, LATENT), lambda i: (0, 0)),
        ],
        out_specs=pl.BlockSpec((_PB, 128), lambda i: (i, 0)),
        out_shape=jax.ShapeDtypeStruct((SEG, 128), jnp.float32),
    )(*([xgt] * 8), w1, b1, w2, b2)


def _sc_body(h_ref, e_ref, out_ref, aggr, ib0, ib1, *rest):
    rb = list(rest[:NB])
    zbuf = rest[NB]
    isem0, isem1 = rest[NB + 1], rest[NB + 2]
    gsem = list(rest[NB + 3:NB + 3 + NB])
    ssem = list(rest[NB + 3 + NB:NB + 3 + 2 * NB])
    c = lax.axis_index("c")
    s = lax.axis_index("s")
    wid = c * NS + s
    extra = (wid < 8).astype(jnp.int32)
    base = wid * CPW + jnp.minimum(wid, 8)
    count = CPW + extra

    def fire_idx(chunk0, ib, sem):
        pltpu.async_copy(e_ref.at[pl.ds(chunk0, IB)], ib, sem)

    def wait_idx(ib, sem):
        pltpu.make_async_copy(e_ref.at[pl.ds(0, IB)], ib, sem).wait()

    fire_idx(base, ib0, isem0)

    @pl.loop(0, ZR)
    def _(i):
        zbuf[i] = jnp.zeros((LATENT,), jnp.float32)

    for k in range(NT // ZR):
        pltpu.sync_copy(zbuf, aggr.at[pl.ds(s * NT + k * ZR, ZR)])
    plsc.subcore_barrier()

    def process16(ib):
        dg = [None] * NB
        ds = [None] * NB
        for t in range(IB + GLA):
            jg = t
            js = t - GLA
            if jg < IB:
                q = jg % NB
                if jg >= NB:
                    ds[q].wait()
                dg[q] = pltpu.async_copy(h_ref.at[ib.at[jg, 0]], rb[q], gsem[q])
            if 0 <= js < IB:
                q = js % NB
                dg[q].wait()
                ds[q] = pltpu.async_copy(rb[q], aggr.at[ib.at[js, 1]],
                                         ssem[q], add=True)
        for js in range(IB - NB, IB):
            ds[js % NB].wait()

    @pl.loop(0, NFULL, step=2)
    def _(b0):
        fire_idx(base + (b0 + 1) * IB, ib1, isem1)
        wait_idx(ib0, isem0)
        process16(ib0)

        @pl.when(b0 + 2 < NFULL)
        def _():
            fire_idx(base + (b0 + 2) * IB, ib0, isem0)

        wait_idx(ib1, isem1)
        process16(ib1)

    rem = count - NFULL * IB
    fire_idx(base + count - IB, ib0, isem0)
    wait_idx(ib0, isem0)
    for j in range(IB):
        @pl.when(j >= IB - rem)
        def _():
            pltpu.async_copy(h_ref.at[ib0.at[j, 0]], rb[0], gsem[0]).wait()
            pltpu.sync_copy(rb[0], aggr.at[ib0.at[j, 1]], add=True)

    plsc.subcore_barrier()
    pltpu.sync_copy(aggr.at[pl.ds(s * NT, NT)],
                    out_ref.at[c].at[pl.ds(s * NT, NT)])


def _sc_aggregate(h, e3):
    mesh = plsc.VectorSubcoreMesh(core_axis_name="c", subcore_axis_name="s",
                                  num_cores=NC, num_subcores=NS)
    f = pl.kernel(
        _sc_body,
        out_type=jax.ShapeDtypeStruct((NC, NP, LATENT), jnp.float32),
        mesh=mesh,
        compiler_params=pltpu.CompilerParams(use_tc_tiling_on_sc=False),
        scratch_types=(
            [pltpu.VMEM_SHARED((NP, LATENT), jnp.float32)]
            + [pltpu.VMEM((IB, 2, CW), jnp.int32)] * 2
            + [pltpu.VMEM((CW, LATENT), jnp.float32)] * NB
            + [pltpu.VMEM((ZR, LATENT), jnp.float32)]
            + [pltpu.SemaphoreType.DMA] * (2 + 2 * NB)
        ),
    )
    return f(h, e3)


_RB = 448


def _update_body(h_ref, p_ref, bw_ref, bb_ref, d1_ref, db1_ref, w2t_ref,
                 sel_ref, db2_ref, o_ref):
    t = _gelu(jnp.dot(h_ref[...], bw_ref[...], preferred_element_type=jnp.float32)
              + bb_ref[...] + p_ref[0] + p_ref[1])
    m = _gelu(jnp.dot(t, d1_ref[...], preferred_element_type=jnp.float32)
              + db1_ref[...])
    o_ref[...] = (jnp.dot(m * w2t_ref[...], sel_ref[...],
                          preferred_element_type=jnp.float32) + db2_ref[...])


def _update_packed(hp, pp, bwp, bbp, d1p, db1p, w2t, sel, db2):
    nb = SEG // _RB
    return pl.pallas_call(
        _update_body,
        grid=(nb,),
        in_specs=[
            pl.BlockSpec((_RB, 128), lambda i: (i, 0)),
            pl.BlockSpec((NC, _RB, 128), lambda i: (0, i, 0)),
            pl.BlockSpec((128, 128), lambda i: (0, 0)),
            pl.BlockSpec((1, 128), lambda i: (0, 0)),
            pl.BlockSpec((128, 128), lambda i: (0, 0)),
            pl.BlockSpec((1, 128), lambda i: (0, 0)),
            pl.BlockSpec((1, 128), lambda i: (0, 0)),
            pl.BlockSpec((128, 8), lambda i: (0, 0)),
            pl.BlockSpec((1, 8), lambda i: (0, 0)),
        ],
        out_specs=pl.BlockSpec((_RB, 8), lambda i: (i, 0)),
        out_shape=jax.ShapeDtypeStruct((SEG, 8), jnp.float32),
    )(hp, pp, bwp, bbp, d1p, db1p, w2t, sel, db2)


def kernel(x, grid, edge_features, proj_w1, proj_b1, proj_w2, proj_b2,
           blk_w, blk_b, dec_w1, dec_b1, dec_w2, dec_b2, edge_index):
    del edge_features
    f32 = jnp.float32
    eye8 = jnp.eye(8, dtype=f32)

    ei_lin = (edge_index.reshape(2, CHUNKS, CW).transpose(1, 0, 2)
              .reshape(2 * CHUNKS, CW))
    e3 = _edge_remap(ei_lin).reshape(CHUNKS, 2, CW)

    xgt = jnp.concatenate([x, grid], axis=1).T
    hc = _project_packed(xgt, proj_w1, proj_b1.reshape(1, LATENT),
                         proj_w2, proj_b2.reshape(1, LATENT))

    part = _sc_aggregate(hc.reshape(NP, LATENT), e3)
    pp = part.reshape(NC, SEG, 128)

    bwp = jnp.kron(eye8, blk_w)
    d1p = jnp.kron(eye8, dec_w1)
    w2t = jnp.tile(dec_w2[:, 0], 8).reshape(1, 128)
    sel = (jnp.arange(128)[:, None] // 16 ==
           jnp.arange(8)[None, :]).astype(f32)
    op = _update_packed(hc, pp, bwp, jnp.tile(blk_b, 8).reshape(1, 128),
                        d1p, jnp.tile(dec_b1, 8).reshape(1, 128),
                        w2t, sel, jnp.broadcast_to(dec_b2, (8,)).reshape(1, 8))
    return op.T.reshape(NP, 1)[:N]


# speedup vs baseline: 79.8513x; 1.1577x over previous
import jax
import jax.numpy as jnp
from jax import lax
from jax.experimental import pallas as pl
from jax.experimental.pallas import tpu as pltpu
from jax.experimental.pallas import tpu_sc as plsc

N = 100000
E = 3200000
LATENT = 16

NC = 2
NS = 16
NW = NC * NS

SEG = 12544
NP = 8 * SEG

CW = 128
CHUNKS = E // CW
CPW = CHUNKS // NW
IB = 16
NFULL = 48
NB = 8
GLA = 4
NT = NP // NS
ZR = 224


def _gelu(t):
    return 0.5 * t * (1.0 + lax.erf(t * (2.0 ** -0.5)))


def _remap(n):
    a = ((n >> 7) * 669) >> 16
    return ((n - a * SEG) << 3) + a


def _remap_body(e_ref, o_ref):
    o_ref[...] = _remap(e_ref[...])


def _edge_remap(ei_lin):
    nb = 25
    rows = 2 * CHUNKS
    return pl.pallas_call(
        _remap_body,
        grid=(nb,),
        in_specs=[pl.BlockSpec((rows // nb, CW), lambda i: (i, 0))],
        out_specs=pl.BlockSpec((rows // nb, CW), lambda i: (i, 0)),
        out_shape=jax.ShapeDtypeStruct((rows, CW), jnp.int32),
    )(ei_lin)


_PB = 896


def _proj_body(*refs):
    xrefs = refs[:8]
    w1_ref, b1_ref, w2_ref, b2_ref, o_ref = refs[8:]
    dn = (((0,), (0,)), ((), ()))
    parts = []
    for a in range(8):
        pre = (lax.dot_general(xrefs[a][...], w1_ref[...], dn,
                               preferred_element_type=jnp.float32)
               + b1_ref[...])
        h2 = (jnp.dot(_gelu(pre), w2_ref[...],
                      preferred_element_type=jnp.float32) + b2_ref[...])
        if a == 7:
            gr = (_PB * pl.program_id(0)
                  + lax.broadcasted_iota(jnp.int32, (_PB, 1), 0))
            h2 = jnp.where(gr < N - 7 * SEG, h2, 0.0)
        parts.append(h2)
    o_ref[...] = jnp.concatenate(parts, axis=1)


def _project_packed(xgt, w1, b1, w2, b2):
    specs = [
        pl.BlockSpec((12, _PB), (lambda i, a=a: (0, (SEG // _PB) * a + i)))
        for a in range(8)
    ]
    return pl.pallas_call(
        _proj_body,
        grid=(SEG // _PB,),
        in_specs=specs + [
            pl.BlockSpec((12, LATENT), lambda i: (0, 0)),
            pl.BlockSpec((1, LATENT), lambda i: (0, 0)),
            pl.BlockSpec((LATENT, LATENT), lambda i: (0, 0)),
            pl.BlockSpec((1, LATENT), lambda i: (0, 0)),
        ],
        out_specs=pl.BlockSpec((_PB, 128), lambda i: (i, 0)),
        out_shape=jax.ShapeDtypeStruct((SEG, 128), jnp.float32),
    )(*([xgt] * 8), w1, b1, w2, b2)


def _sc_body(h_ref, e_ref, out_ref, aggr, ib0, ib1, *rest):
    rb = list(rest[:NB])
    zbuf = rest[NB]
    isem0, isem1 = rest[NB + 1], rest[NB + 2]
    gsem = list(rest[NB + 3:NB + 3 + NB])
    ssem = list(rest[NB + 3 + NB:NB + 3 + 2 * NB])
    c = lax.axis_index("c")
    s = lax.axis_index("s")
    wid = c * NS + s
    extra = (wid < 8).astype(jnp.int32)
    base = wid * CPW + jnp.minimum(wid, 8)
    count = CPW + extra

    def fire_idx(chunk0, ib, sem):
        pltpu.async_copy(e_ref.at[pl.ds(chunk0, IB)], ib, sem)

    def wait_idx(ib, sem):
        pltpu.make_async_copy(e_ref.at[pl.ds(0, IB)], ib, sem).wait()

    def remap_idx(ib):
        @pl.loop(0, IB)
        def _(j):
            for u in range(2):
                for v in range(CW // 16):
                    nv = ib[j, u, pl.ds(v * 16, 16)]
                    ib[j, u, pl.ds(v * 16, 16)] = _remap(nv)

    fire_idx(base, ib0, isem0)

    @pl.loop(0, ZR)
    def _(i):
        zbuf[i] = jnp.zeros((LATENT,), jnp.float32)

    for k in range(NT // ZR):
        pltpu.sync_copy(zbuf, aggr.at[pl.ds(s * NT + k * ZR, ZR)])
    plsc.subcore_barrier()

    def process16(ib):
        dg = [None] * NB
        ds = [None] * NB
        for t in range(IB + GLA):
            jg = t
            js = t - GLA
            if jg < IB:
                q = jg % NB
                if jg >= NB:
                    ds[q].wait()
                dg[q] = pltpu.async_copy(h_ref.at[ib.at[jg, 0]], rb[q], gsem[q])
            if 0 <= js < IB:
                q = js % NB
                dg[q].wait()
                ds[q] = pltpu.async_copy(rb[q], aggr.at[ib.at[js, 1]],
                                         ssem[q], add=True)
        for js in range(IB - NB, IB):
            ds[js % NB].wait()

    @pl.loop(0, NFULL, step=2)
    def _(b0):
        fire_idx(base + (b0 + 1) * IB, ib1, isem1)
        wait_idx(ib0, isem0)
        remap_idx(ib0)
        process16(ib0)

        @pl.when(b0 + 2 < NFULL)
        def _():
            fire_idx(base + (b0 + 2) * IB, ib0, isem0)

        wait_idx(ib1, isem1)
        remap_idx(ib1)
        process16(ib1)

    rem = count - NFULL * IB
    fire_idx(base + count - IB, ib0, isem0)
    wait_idx(ib0, isem0)
    remap_idx(ib0)
    for j in range(IB):
        @pl.when(j >= IB - rem)
        def _():
            pltpu.async_copy(h_ref.at[ib0.at[j, 0]], rb[0], gsem[0]).wait()
            pltpu.sync_copy(rb[0], aggr.at[ib0.at[j, 1]], add=True)

    plsc.subcore_barrier()
    pltpu.sync_copy(aggr.at[pl.ds(s * NT, NT)],
                    out_ref.at[c].at[pl.ds(s * NT, NT)])


def _sc_aggregate(h, e3):
    mesh = plsc.VectorSubcoreMesh(core_axis_name="c", subcore_axis_name="s",
                                  num_cores=NC, num_subcores=NS)
    f = pl.kernel(
        _sc_body,
        out_type=jax.ShapeDtypeStruct((NC, NP, LATENT), jnp.float32),
        mesh=mesh,
        compiler_params=pltpu.CompilerParams(use_tc_tiling_on_sc=False),
        scratch_types=(
            [pltpu.VMEM_SHARED((NP, LATENT), jnp.float32)]
            + [pltpu.VMEM((IB, 2, CW), jnp.int32)] * 2
            + [pltpu.VMEM((CW, LATENT), jnp.float32)] * NB
            + [pltpu.VMEM((ZR, LATENT), jnp.float32)]
            + [pltpu.SemaphoreType.DMA] * (2 + 2 * NB)
        ),
    )
    return f(h, e3)


_RB = 1792


def _update_body(h_ref, p_ref, bw_ref, bb_ref, d1_ref, db1_ref, w2t_ref,
                 sel_ref, db2_ref, o_ref):
    t = _gelu(jnp.dot(h_ref[...], bw_ref[...], preferred_element_type=jnp.float32)
              + bb_ref[...] + p_ref[0] + p_ref[1])
    m = _gelu(jnp.dot(t, d1_ref[...], preferred_element_type=jnp.float32)
              + db1_ref[...])
    o_ref[...] = (lax.dot_general(sel_ref[...], m * w2t_ref[...],
                                  (((0,), (1,)), ((), ())),
                                  preferred_element_type=jnp.float32)
                  + db2_ref[...])


def _update_packed(hp, pp, bwp, bbp, d1p, db1p, w2t, sel, db2):
    nb = SEG // _RB
    return pl.pallas_call(
        _update_body,
        grid=(nb,),
        in_specs=[
            pl.BlockSpec((_RB, 128), lambda i: (i, 0)),
            pl.BlockSpec((NC, _RB, 128), lambda i: (0, i, 0)),
            pl.BlockSpec((128, 128), lambda i: (0, 0)),
            pl.BlockSpec((1, 128), lambda i: (0, 0)),
            pl.BlockSpec((128, 128), lambda i: (0, 0)),
            pl.BlockSpec((1, 128), lambda i: (0, 0)),
            pl.BlockSpec((1, 128), lambda i: (0, 0)),
            pl.BlockSpec((128, 8), lambda i: (0, 0)),
            pl.BlockSpec((8, 1), lambda i: (0, 0)),
        ],
        out_specs=pl.BlockSpec((8, _RB), lambda i: (0, i)),
        out_shape=jax.ShapeDtypeStruct((8, SEG), jnp.float32),
    )(hp, pp, bwp, bbp, d1p, db1p, w2t, sel, db2)


def kernel(x, grid, edge_features, proj_w1, proj_b1, proj_w2, proj_b2,
           blk_w, blk_b, dec_w1, dec_b1, dec_w2, dec_b2, edge_index):
    del edge_features
    f32 = jnp.float32
    eye8 = jnp.eye(8, dtype=f32)

    e3 = edge_index.reshape(2, CHUNKS, CW).transpose(1, 0, 2)

    xgt = jnp.concatenate([x, grid], axis=1).T
    hc = _project_packed(xgt, proj_w1, proj_b1.reshape(1, LATENT),
                         proj_w2, proj_b2.reshape(1, LATENT))

    part = _sc_aggregate(hc.reshape(NP, LATENT), e3)
    pp = part.reshape(NC, SEG, 128)

    bwp = jnp.kron(eye8, blk_w)
    d1p = jnp.kron(eye8, dec_w1)
    w2t = jnp.tile(dec_w2[:, 0], 8).reshape(1, 128)
    sel = (jnp.arange(128)[:, None] // 16 ==
           jnp.arange(8)[None, :]).astype(f32)
    op = _update_packed(hc, pp, bwp, jnp.tile(blk_b, 8).reshape(1, 128),
                        d1p, jnp.tile(dec_b1, 8).reshape(1, 128),
                        w2t, sel, jnp.broadcast_to(dec_b2, (8,)).reshape(8, 1))
    return op.reshape(NP, 1)[:N]


# speedup vs baseline: 81.2651x; 1.0177x over previous
import jax
import jax.numpy as jnp
from jax import lax
from jax.experimental import pallas as pl
from jax.experimental.pallas import tpu as pltpu
from jax.experimental.pallas import tpu_sc as plsc

N = 100000
E = 3200000
LATENT = 16

NC = 2
NS = 16
NW = NC * NS

SEG = 12544
NP = 8 * SEG

CW = 128
CHUNKS = E // CW
CPW = CHUNKS // NW
IB = 16
NFULL = 48
NB = 8
GLA = 4
NT = NP // NS
ZR = 224


def _gelu(t):
    return 0.5 * t * (1.0 + lax.erf(t * (2.0 ** -0.5)))


def _remap(n):
    a = ((n >> 7) * 669) >> 16
    return ((n - a * SEG) << 3) + a


def _remap_body(e_ref, o_ref):
    o_ref[...] = _remap(e_ref[...])


def _edge_remap(ei_lin):
    nb = 25
    rows = 2 * CHUNKS
    return pl.pallas_call(
        _remap_body,
        grid=(nb,),
        in_specs=[pl.BlockSpec((rows // nb, CW), lambda i: (i, 0))],
        out_specs=pl.BlockSpec((rows // nb, CW), lambda i: (i, 0)),
        out_shape=jax.ShapeDtypeStruct((rows, CW), jnp.int32),
    )(ei_lin)


_PB = 896


def _proj_body(*refs):
    xrefs = refs[:8]
    w1_ref, b1_ref, w2p_ref, b2t_ref, o_ref = refs[8:]
    dn = (((0,), (0,)), ((), ()))
    acc = b2t_ref[...]
    for a in range(8):
        pre = (lax.dot_general(xrefs[a][...], w1_ref[...], dn,
                               preferred_element_type=jnp.float32)
               + b1_ref[...])
        g = _gelu(pre)
        if a == 7:
            gr = (_PB * pl.program_id(0)
                  + lax.broadcasted_iota(jnp.int32, (_PB, 1), 0))
            g = jnp.where(gr < N - 7 * SEG, g, 0.0)
        acc = acc + jnp.dot(g, w2p_ref[16 * a:16 * (a + 1), :],
                            preferred_element_type=jnp.float32)
    o_ref[...] = acc


def _project_packed(xgt, w1, b1, w2p, b2t):
    specs = [
        pl.BlockSpec((12, _PB), (lambda i, a=a: (0, (SEG // _PB) * a + i)))
        for a in range(8)
    ]
    return pl.pallas_call(
        _proj_body,
        grid=(SEG // _PB,),
        in_specs=specs + [
            pl.BlockSpec((12, LATENT), lambda i: (0, 0)),
            pl.BlockSpec((1, LATENT), lambda i: (0, 0)),
            pl.BlockSpec((128, 128), lambda i: (0, 0)),
            pl.BlockSpec((1, 128), lambda i: (0, 0)),
        ],
        out_specs=pl.BlockSpec((_PB, 128), lambda i: (i, 0)),
        out_shape=jax.ShapeDtypeStruct((SEG, 128), jnp.float32),
    )(*([xgt] * 8), w1, b1, w2p, b2t)


def _sc_body(h_ref, e_ref, out_ref, aggr, ib0, ib1, *rest):
    rb = list(rest[:NB])
    zbuf = rest[NB]
    isem0, isem1 = rest[NB + 1], rest[NB + 2]
    gsem = list(rest[NB + 3:NB + 3 + NB])
    ssem = list(rest[NB + 3 + NB:NB + 3 + 2 * NB])
    c = lax.axis_index("c")
    s = lax.axis_index("s")
    wid = c * NS + s
    extra = (wid < 8).astype(jnp.int32)
    base = wid * CPW + jnp.minimum(wid, 8)
    count = CPW + extra

    def fire_idx(chunk0, ib, sem):
        pltpu.async_copy(e_ref.at[pl.ds(chunk0, IB)], ib, sem)

    def wait_idx(ib, sem):
        pltpu.make_async_copy(e_ref.at[pl.ds(0, IB)], ib, sem).wait()

    def remap_idx(ib):
        @pl.loop(0, IB)
        def _(j):
            for u in range(2):
                for v in range(CW // 16):
                    nv = ib[j, u, pl.ds(v * 16, 16)]
                    ib[j, u, pl.ds(v * 16, 16)] = _remap(nv)

    fire_idx(base, ib0, isem0)

    @pl.loop(0, ZR)
    def _(i):
        zbuf[i] = jnp.zeros((LATENT,), jnp.float32)

    for k in range(NT // ZR):
        pltpu.sync_copy(zbuf, aggr.at[pl.ds(s * NT + k * ZR, ZR)])
    plsc.subcore_barrier()

    def process16(ib):
        dg = [None] * NB
        ds = [None] * NB
        for t in range(IB + GLA):
            jg = t
            js = t - GLA
            if jg < IB:
                q = jg % NB
                if jg >= NB:
                    ds[q].wait()
                dg[q] = pltpu.async_copy(h_ref.at[ib.at[jg, 0]], rb[q], gsem[q])
            if 0 <= js < IB:
                q = js % NB
                dg[q].wait()
                ds[q] = pltpu.async_copy(rb[q], aggr.at[ib.at[js, 1]],
                                         ssem[q], add=True)
        for js in range(IB - NB, IB):
            ds[js % NB].wait()

    @pl.loop(0, NFULL, step=2)
    def _(b0):
        fire_idx(base + (b0 + 1) * IB, ib1, isem1)
        wait_idx(ib0, isem0)
        remap_idx(ib0)
        process16(ib0)

        @pl.when(b0 + 2 < NFULL)
        def _():
            fire_idx(base + (b0 + 2) * IB, ib0, isem0)

        wait_idx(ib1, isem1)
        remap_idx(ib1)
        process16(ib1)

    rem = count - NFULL * IB
    fire_idx(base + count - IB, ib0, isem0)
    wait_idx(ib0, isem0)
    remap_idx(ib0)
    for j in range(IB):
        @pl.when(j >= IB - rem)
        def _():
            pltpu.async_copy(h_ref.at[ib0.at[j, 0]], rb[0], gsem[0]).wait()
            pltpu.sync_copy(rb[0], aggr.at[ib0.at[j, 1]], add=True)

    plsc.subcore_barrier()
    pltpu.sync_copy(aggr.at[pl.ds(s * NT, NT)],
                    out_ref.at[c].at[pl.ds(s * NT, NT)])


def _sc_aggregate(h, e3):
    mesh = plsc.VectorSubcoreMesh(core_axis_name="c", subcore_axis_name="s",
                                  num_cores=NC, num_subcores=NS)
    f = pl.kernel(
        _sc_body,
        out_type=jax.ShapeDtypeStruct((NC, NP, LATENT), jnp.float32),
        mesh=mesh,
        compiler_params=pltpu.CompilerParams(use_tc_tiling_on_sc=False),
        scratch_types=(
            [pltpu.VMEM_SHARED((NP, LATENT), jnp.float32)]
            + [pltpu.VMEM((IB, 2, CW), jnp.int32)] * 2
            + [pltpu.VMEM((CW, LATENT), jnp.float32)] * NB
            + [pltpu.VMEM((ZR, LATENT), jnp.float32)]
            + [pltpu.SemaphoreType.DMA] * (2 + 2 * NB)
        ),
    )
    return f(h, e3)


_RB = 1792


def _update_body(h_ref, p_ref, bw_ref, bb_ref, d1_ref, db1_ref, w2t_ref,
                 sel_ref, db2_ref, o_ref):
    t = _gelu(jnp.dot(h_ref[...], bw_ref[...], preferred_element_type=jnp.float32)
              + bb_ref[...] + p_ref[0] + p_ref[1])
    m = _gelu(jnp.dot(t, d1_ref[...], preferred_element_type=jnp.float32)
              + db1_ref[...])
    o_ref[...] = (lax.dot_general(sel_ref[...], m * w2t_ref[...],
                                  (((0,), (1,)), ((), ())),
                                  preferred_element_type=jnp.float32)
                  + db2_ref[...])


def _update_packed(hp, pp, bwp, bbp, d1p, db1p, w2t, sel, db2):
    nb = SEG // _RB
    return pl.pallas_call(
        _update_body,
        grid=(nb,),
        in_specs=[
            pl.BlockSpec((_RB, 128), lambda i: (i, 0)),
            pl.BlockSpec((NC, _RB, 128), lambda i: (0, i, 0)),
            pl.BlockSpec((128, 128), lambda i: (0, 0)),
            pl.BlockSpec((1, 128), lambda i: (0, 0)),
            pl.BlockSpec((128, 128), lambda i: (0, 0)),
            pl.BlockSpec((1, 128), lambda i: (0, 0)),
            pl.BlockSpec((1, 128), lambda i: (0, 0)),
            pl.BlockSpec((128, 8), lambda i: (0, 0)),
            pl.BlockSpec((8, 1), lambda i: (0, 0)),
        ],
        out_specs=pl.BlockSpec((8, _RB), lambda i: (0, i)),
        out_shape=jax.ShapeDtypeStruct((8, SEG), jnp.float32),
    )(hp, pp, bwp, bbp, d1p, db1p, w2t, sel, db2)


def kernel(x, grid, edge_features, proj_w1, proj_b1, proj_w2, proj_b2,
           blk_w, blk_b, dec_w1, dec_b1, dec_w2, dec_b2, edge_index):
    del edge_features
    f32 = jnp.float32
    eye8 = jnp.eye(8, dtype=f32)

    e3 = edge_index.reshape(2, CHUNKS, CW).transpose(1, 0, 2)

    xgt = jnp.concatenate([x, grid], axis=1).T
    hc = _project_packed(xgt, proj_w1, proj_b1.reshape(1, LATENT),
                         jnp.kron(eye8, proj_w2),
                         jnp.tile(proj_b2, 8).reshape(1, 128))

    part = _sc_aggregate(hc.reshape(NP, LATENT), e3)
    pp = part.reshape(NC, SEG, 128)

    bwp = jnp.kron(eye8, blk_w)
    d1p = jnp.kron(eye8, dec_w1)
    w2t = jnp.tile(dec_w2[:, 0], 8).reshape(1, 128)
    sel = (jnp.arange(128)[:, None] // 16 ==
           jnp.arange(8)[None, :]).astype(f32)
    op = _update_packed(hc, pp, bwp, jnp.tile(blk_b, 8).reshape(1, 128),
                        d1p, jnp.tile(dec_b1, 8).reshape(1, 128),
                        w2t, sel, jnp.broadcast_to(dec_b2, (8,)).reshape(8, 1))
    return op.reshape(NP, 1)[:N]
